# baseline (speedup 1.0000x reference)
import jax
import jax.numpy as jnp
from jax import lax
from jax.experimental import pallas as pl
from jax.experimental.pallas import tpu as pltpu

E = 64
RANK = 16
ER = E * RANK
D_MODEL = 1024
D_FF = 2048
ALPHA = 2.0
S = 2048

FC = 512
N_FC = D_FF // FC
KC = 512
N_KC = D_FF // KC
BF = jnp.bfloat16


def _dot_t(a, b):
    return lax.dot_general(a, b, (((1,), (1,)), ((), ())),
                           preferred_element_type=jnp.float32)


def _dot(a, b):
    return lax.dot_general(a, b, (((1,), (0,)), ((), ())),
                           preferred_element_type=jnp.float32)


def _router_body(x_ref, wr_ref, sel_ref, coef_ref, xb_ref):
    x = x_ref[...]
    logits = _dot_t(x, wr_ref[...])
    m = jnp.max(logits, axis=1, keepdims=True)
    eids = lax.broadcasted_iota(jnp.int32, logits.shape, 1)
    sel_ref[...] = jnp.min(jnp.where(logits >= m, eids, E), axis=1,
                           keepdims=True)
    p = jnp.exp(m - jnp.max(m))
    coef_ref[...] = p / jnp.sum(p)
    xb_ref[...] = x.astype(BF)


def _upgate_body(xb_ref, sel_ref, au_ref, ag_ref, wu_ref, wg_ref,
                 bu_ref, bg_ref, hid_ref, pmu_s, pmg_s):
    c = pl.program_id(0)

    @pl.when(c == 0)
    def _():
        xb = xb_ref[...]
        sel = sel_ref[...]
        pu = _dot_t(xb, au_ref[...])
        pg = _dot_t(xb, ag_ref[...])
        mask = (lax.broadcasted_iota(jnp.int32, pu.shape, 1) // RANK) == sel
        pmu_s[...] = jnp.where(mask, pu, 0.0).astype(BF)
        pmg_s[...] = jnp.where(mask, pg, 0.0).astype(BF)

    xb = xb_ref[...]
    h1 = _dot_t(xb, wu_ref[...].astype(BF))
    h3 = _dot_t(xb, wg_ref[...].astype(BF))
    l1 = _dot(pmu_s[...], bu_ref[...])
    l3 = _dot(pmg_s[...], bg_ref[...])
    a = h1 + ALPHA * l1
    b = h3 + ALPHA * l3
    hid_ref[...] = (a * jax.nn.sigmoid(a) * b).astype(BF)


def _down_body(hid_ref, sel_ref, coef_ref, ad_ref, wd_ref, bd_ref,
               out_ref, wacc_s, qacc_s):
    c = pl.program_id(0)
    hb = hid_ref[...]
    wpart = _dot_t(hb, wd_ref[...].astype(BF))
    qpart = _dot_t(hb, ad_ref[...].astype(BF))

    @pl.when(c == 0)
    def _():
        wacc_s[...] = wpart.astype(BF)
        qacc_s[...] = qpart.astype(BF)

    @pl.when(jnp.logical_and(c > 0, c < N_KC - 1))
    def _():
        wacc_s[...] = (wacc_s[...].astype(jnp.float32) + wpart).astype(BF)
        qacc_s[...] = (qacc_s[...].astype(jnp.float32) + qpart).astype(BF)

    @pl.when(c == N_KC - 1)
    def _():
        for r in range(2):
            rs = slice(r * (S // 2), (r + 1) * (S // 2))
            base = wacc_s[rs, :].astype(jnp.float32) + wpart[rs]
            qd = qacc_s[rs, :].astype(jnp.float32) + qpart[rs]
            sel = sel_ref[rs, :]
            mask = (lax.broadcasted_iota(jnp.int32, qd.shape, 1)
                    // RANK) == sel
            qm = jnp.where(mask, qd, 0.0).astype(BF)
            l2 = _dot(qm, bd_ref[...])
            out_ref[rs, :] = coef_ref[rs, :] * (base + ALPHA * l2)


@jax.jit
def _run(x, W_up, W_gate_proj, W_down, W_router,
         up_A, up_B, down_A, down_B, gate_A, gate_B):
    sel, coef, xb = pl.pallas_call(
        _router_body,
        out_shape=(jax.ShapeDtypeStruct((S, 1), jnp.int32),
                   jax.ShapeDtypeStruct((S, 1), jnp.float32),
                   jax.ShapeDtypeStruct((S, D_MODEL), BF)),
    )(x, W_router)

    au = up_A.reshape(ER, D_MODEL).astype(BF)
    ag = gate_A.reshape(ER, D_MODEL).astype(BF)
    ad = down_A.reshape(ER, D_FF)
    bu = up_B.transpose(0, 2, 1).reshape(ER, D_FF).astype(BF)
    bg = gate_B.transpose(0, 2, 1).reshape(ER, D_FF).astype(BF)
    bd = down_B.transpose(0, 2, 1).reshape(ER, D_MODEL).astype(BF)

    full = lambda c: (0, 0)
    arb = pltpu.CompilerParams(dimension_semantics=("arbitrary",))

    hidden = pl.pallas_call(
        _upgate_body,
        grid=(N_FC,),
        in_specs=[
            pl.BlockSpec((S, D_MODEL), full),
            pl.BlockSpec((S, 1), full),
            pl.BlockSpec((ER, D_MODEL), full),
            pl.BlockSpec((ER, D_MODEL), full),
            pl.BlockSpec((FC, D_MODEL), lambda c: (c, 0)),
            pl.BlockSpec((FC, D_MODEL), lambda c: (c, 0)),
            pl.BlockSpec((ER, FC), lambda c: (0, c)),
            pl.BlockSpec((ER, FC), lambda c: (0, c)),
        ],
        out_specs=pl.BlockSpec((S, FC), lambda c: (0, c)),
        out_shape=jax.ShapeDtypeStruct((S, D_FF), BF),
        scratch_shapes=[
            pltpu.VMEM((S, ER), BF),
            pltpu.VMEM((S, ER), BF),
        ],
        compiler_params=arb,
    )(xb, sel, au, ag, W_up, W_gate_proj, bu, bg)

    out = pl.pallas_call(
        _down_body,
        grid=(N_KC,),
        in_specs=[
            pl.BlockSpec((S, KC), lambda c: (0, c)),
            pl.BlockSpec((S, 1), full),
            pl.BlockSpec((S, 1), full),
            pl.BlockSpec((ER, KC), lambda c: (0, c)),
            pl.BlockSpec((D_MODEL, KC), lambda c: (0, c)),
            pl.BlockSpec((ER, D_MODEL), full),
        ],
        out_specs=pl.BlockSpec((S, D_MODEL), full),
        out_shape=jax.ShapeDtypeStruct((S, D_MODEL), jnp.float32),
        scratch_shapes=[
            pltpu.VMEM((S, D_MODEL), BF),
            pltpu.VMEM((S, ER), BF),
        ],
        compiler_params=arb,
    )(hidden, sel, coef, ad, W_down, bd)
    return out


def kernel(inputs, W_up, W_gate_proj, W_down, W_router,
           up_A, up_B, down_A, down_B, gate_A, gate_B):
    x = inputs.reshape(S, D_MODEL)
    out = _run(x, W_up, W_gate_proj, W_down, W_router,
               up_A, up_B, down_A, down_B, gate_A, gate_B)
    return out.reshape(1, S, D_MODEL)


# speedup vs baseline: 1.0693x; 1.0693x over previous
import jax
import jax.numpy as jnp
from jax import lax
from jax.experimental import pallas as pl
from jax.experimental.pallas import tpu as pltpu

E = 64
RANK = 16
ER = E * RANK
D_MODEL = 1024
D_FF = 2048
ALPHA = 2.0
S = 2048

FC = 256
N_FC = D_FF // FC
KC = 256
N_KC = D_FF // KC
BF = jnp.bfloat16


def _dot_t(a, b):
    return lax.dot_general(a, b, (((1,), (1,)), ((), ())),
                           preferred_element_type=jnp.float32)


def _dot(a, b):
    return lax.dot_general(a, b, (((1,), (0,)), ((), ())),
                           preferred_element_type=jnp.float32)


def _router_body(x_ref, wr_ref, sel_ref, coef_ref, xb_ref):
    x = x_ref[...]
    logits = _dot_t(x, wr_ref[...])
    m = jnp.max(logits, axis=1, keepdims=True)
    eids = lax.broadcasted_iota(jnp.int32, logits.shape, 1)
    sel_ref[...] = jnp.min(jnp.where(logits >= m, eids, E), axis=1,
                           keepdims=True)
    p = jnp.exp(m - jnp.max(m))
    coef_ref[...] = p / jnp.sum(p)
    xb_ref[...] = x.astype(BF)


def _upgate_body(xb_ref, sel_ref, au_ref, ag_ref, wu_ref, wg_ref,
                 bu_ref, bg_ref, hid_ref, pmu_s, pmg_s):
    c = pl.program_id(0)

    @pl.when(c == 0)
    def _():
        xb = xb_ref[...]
        sel = sel_ref[...]
        pu = _dot_t(xb, au_ref[...].astype(BF))
        pg = _dot_t(xb, ag_ref[...].astype(BF))
        mask = (lax.broadcasted_iota(jnp.int32, pu.shape, 1) // RANK) == sel
        pmu_s[...] = jnp.where(mask, pu, 0.0).astype(BF)
        pmg_s[...] = jnp.where(mask, pg, 0.0).astype(BF)

    xb = xb_ref[...]
    h1 = _dot_t(xb, wu_ref[...].astype(BF))
    h3 = _dot_t(xb, wg_ref[...].astype(BF))
    l1 = _dot(pmu_s[...], bu_ref[...])
    l3 = _dot(pmg_s[...], bg_ref[...])
    a = h1 + ALPHA * l1
    b = h3 + ALPHA * l3
    hid_ref[...] = (a * jax.nn.sigmoid(a) * b).astype(BF)


def _down_body(hid_ref, sel_ref, coef_ref, ad_ref, wd_ref, bd_ref,
               out_ref, wacc_s, qacc_s):
    c = pl.program_id(0)

    @pl.when(c == 0)
    def _():
        wacc_s[...] = jnp.zeros_like(wacc_s)
        qacc_s[...] = jnp.zeros_like(qacc_s)

    @pl.when(c < N_KC)
    def _():
        hb = hid_ref[...]
        wacc_s[...] += _dot_t(hb, wd_ref[...].astype(BF))
        qacc_s[...] += _dot_t(hb, ad_ref[...].astype(BF))

    @pl.when(c == N_KC)
    def _():
        sel = sel_ref[...]
        qd = qacc_s[...]
        mask = (lax.broadcasted_iota(jnp.int32, qd.shape, 1) // RANK) == sel
        qm = jnp.where(mask, qd, 0.0).astype(BF)
        l2 = _dot(qm, bd_ref[...])
        out_ref[...] = coef_ref[...] * (wacc_s[...] + ALPHA * l2)


@jax.jit
def _run(x, W_up, W_gate_proj, W_down, W_router,
         up_A, up_B, down_A, down_B, gate_A, gate_B):
    sel, coef, xb = pl.pallas_call(
        _router_body,
        out_shape=(jax.ShapeDtypeStruct((S, 1), jnp.int32),
                   jax.ShapeDtypeStruct((S, 1), jnp.float32),
                   jax.ShapeDtypeStruct((S, D_MODEL), BF)),
    )(x, W_router)

    au = up_A.reshape(ER, D_MODEL)
    ag = gate_A.reshape(ER, D_MODEL)
    ad = down_A.reshape(ER, D_FF)
    bu = up_B.astype(BF).transpose(0, 2, 1).reshape(ER, D_FF)
    bg = gate_B.astype(BF).transpose(0, 2, 1).reshape(ER, D_FF)
    bd = down_B.astype(BF).transpose(0, 2, 1).reshape(ER, D_MODEL)

    full = lambda c: (0, 0)
    arb = pltpu.CompilerParams(dimension_semantics=("arbitrary",))

    hidden = pl.pallas_call(
        _upgate_body,
        grid=(N_FC,),
        in_specs=[
            pl.BlockSpec((S, D_MODEL), full),
            pl.BlockSpec((S, 1), full),
            pl.BlockSpec((ER, D_MODEL), full),
            pl.BlockSpec((ER, D_MODEL), full),
            pl.BlockSpec((FC, D_MODEL), lambda c: (c, 0)),
            pl.BlockSpec((FC, D_MODEL), lambda c: (c, 0)),
            pl.BlockSpec((ER, FC), lambda c: (0, c)),
            pl.BlockSpec((ER, FC), lambda c: (0, c)),
        ],
        out_specs=pl.BlockSpec((S, FC), lambda c: (0, c)),
        out_shape=jax.ShapeDtypeStruct((S, D_FF), BF),
        scratch_shapes=[
            pltpu.VMEM((S, ER), BF),
            pltpu.VMEM((S, ER), BF),
        ],
        compiler_params=arb,
    )(xb, sel, au, ag, W_up, W_gate_proj, bu, bg)

    cc = lambda c: jnp.minimum(c, N_KC - 1)
    out = pl.pallas_call(
        _down_body,
        grid=(N_KC + 1,),
        in_specs=[
            pl.BlockSpec((S, KC), lambda c: (0, cc(c))),
            pl.BlockSpec((S, 1), full),
            pl.BlockSpec((S, 1), full),
            pl.BlockSpec((ER, KC), lambda c: (0, cc(c))),
            pl.BlockSpec((D_MODEL, KC), lambda c: (0, cc(c))),
            pl.BlockSpec((ER, D_MODEL), full),
        ],
        out_specs=pl.BlockSpec((S, D_MODEL), full),
        out_shape=jax.ShapeDtypeStruct((S, D_MODEL), jnp.float32),
        scratch_shapes=[
            pltpu.VMEM((S, D_MODEL), jnp.float32),
            pltpu.VMEM((S, ER), jnp.float32),
        ],
        compiler_params=arb,
    )(hidden, sel, coef, ad, W_down, bd)
    return out


def kernel(inputs, W_up, W_gate_proj, W_down, W_router,
           up_A, up_B, down_A, down_B, gate_A, gate_B):
    x = inputs.reshape(S, D_MODEL)
    out = _run(x, W_up, W_gate_proj, W_down, W_router,
               up_A, up_B, down_A, down_B, gate_A, gate_B)
    return out.reshape(1, S, D_MODEL)
